# baseline (speedup 1.0000x reference)
import functools

import jax
import jax.numpy as jnp
from jax import lax
from jax.experimental import pallas as pl
from jax.experimental.pallas import tpu as pltpu
from jax.experimental.pallas import tpu_sc as plsc

try:
    _INFO = plsc.get_sparse_core_info()
    _NC, _NS = _INFO.num_cores, _INFO.num_subcores
except Exception:
    _NC, _NS = 2, 16
_NW = _NC * _NS

_L = 16
_TJ = 8
_TI = 128


def kernel(x, weight):
    B, S = x.shape
    V, D = weight.shape
    N = B * S
    njb = S // _TJ
    nib = B // _TI
    nu = njb * nib
    ib_per_w = nib // _NW
    blk = _TJ * _TI
    obuf = D * blk

    tab_size = V * D + D - 1
    tab_size += (-tab_size) % 8
    wflat = jnp.concatenate(
        [weight.reshape(-1), jnp.zeros((tab_size - V * D,), jnp.float32)]
    )

    mesh = plsc.VectorSubcoreMesh(
        core_axis_name="c", subcore_axis_name="s", num_cores=_NC, num_subcores=_NS
    )

    @functools.partial(
        pl.kernel,
        out_type=jax.ShapeDtypeStruct((N * D,), jnp.float32),
        mesh=mesh,
        scratch_types=[
            pltpu.VMEM((tab_size,), jnp.float32),
            pltpu.VMEM((_TI, S), jnp.int32),
            pltpu.VMEM((_TI, S), jnp.int32),
            pltpu.VMEM((obuf,), jnp.float32),
            pltpu.VMEM((obuf,), jnp.float32),
            pltpu.SemaphoreType.DMA,
            pltpu.SemaphoreType.DMA,
        ],
        compiler_params=pltpu.CompilerParams(
            use_tc_tiling_on_sc=False, needs_layout_passes=False
        ),
    )
    def emb(x_hbm, w_hbm, out_hbm, tab_v, xr0, xr1, oa, ob, soa, sob):
        wid = lax.axis_index("s") * _NC + lax.axis_index("c")
        ib0w = wid * ib_per_w
        pltpu.sync_copy(w_hbm, tab_v)
        iota = lax.iota(jnp.int32, _L)
        xrs, os_, sos = [xr0, xr1], [oa, ob], [soa, sob]

        def compute(b, jb):
            def k_body(k2, carry):
                for sub in range(2):
                    k = k2 * 2 + sub
                    jr = k // _TJ
                    kk = k % _TJ
                    idxv = plsc.load_gather(
                        xrs[b],
                        [iota + kk * _L, jnp.broadcast_to(jb * _TJ, (_L,)) + jr],
                    )
                    a = idxv * D
                    vals = [plsc.load_gather(tab_v, [a + d]) for d in range(D)]
                    for d in range(D):
                        os_[b][pl.ds(d * blk + k * _L, _L)] = vals[d]
                return carry

            lax.fori_loop(0, blk // _L // 2, k_body, 0)

        def fire_out(b, jb, ib):
            u = jb * nib + ib
            for d in range(D):
                pltpu.async_copy(
                    os_[b].at[pl.ds(d * blk, blk)],
                    out_hbm.at[pl.ds(d * (nu * blk) + u * blk, blk)],
                    sos[b],
                )

        def drain_out(b):
            pltpu.make_async_copy(
                out_hbm.at[pl.ds(0, obuf)], os_[b], sos[b]
            ).wait()

        for half in range(ib_per_w // 2):
            ib_a = ib0w + 2 * half
            for b in range(2):
                pltpu.sync_copy(
                    x_hbm.at[pl.ds((ib_a + b) * _TI, _TI), pl.ds(0, S)], xrs[b]
                )

            def jb_body(jb, carry):
                for b in range(2):
                    @pl.when(jnp.logical_or(jb >= 1, half >= 1))
                    def _():
                        drain_out(b)

                    compute(b, jb)
                    fire_out(b, jb, ib_a + b)
                return carry

            lax.fori_loop(0, njb, jb_body, 0)

        for b in range(2):
            drain_out(b)

    out = emb(x, wflat)
    return (
        out.reshape(D, njb, nib, _TJ, _TI)
        .transpose(2, 4, 1, 3, 0)
        .reshape(B, S, D)
    )


# speedup vs baseline: 1.1015x; 1.1015x over previous
import functools

import jax
import jax.numpy as jnp
from jax import lax
from jax.experimental import pallas as pl
from jax.experimental.pallas import tpu as pltpu
from jax.experimental.pallas import tpu_sc as plsc

try:
    _INFO = plsc.get_sparse_core_info()
    _NC, _NS = _INFO.num_cores, _INFO.num_subcores
except Exception:
    _NC, _NS = 2, 16
_NW = _NC * _NS

_L = 16
_TJ = 8
_TI = 128


def kernel(x, weight):
    B, S = x.shape
    V, D = weight.shape
    N = B * S
    njb = S // _TJ
    nib = B // _TI
    nu = njb * nib
    ib_per_w = nib // _NW
    blk = _TJ * _TI
    obuf = D * blk

    tab_size = V * D + D - 1
    tab_size += (-tab_size) % 8
    wflat = jnp.concatenate(
        [weight.reshape(-1), jnp.zeros((tab_size - V * D,), jnp.float32)]
    )

    mesh = plsc.VectorSubcoreMesh(
        core_axis_name="c", subcore_axis_name="s", num_cores=_NC, num_subcores=_NS
    )

    @functools.partial(
        pl.kernel,
        out_type=jax.ShapeDtypeStruct((N * D,), jnp.float32),
        mesh=mesh,
        scratch_types=[
            pltpu.VMEM((tab_size,), jnp.float32),
            pltpu.VMEM((_TI * S,), jnp.int32),
            pltpu.VMEM((_TI * S,), jnp.int32),
            pltpu.VMEM((obuf,), jnp.float32),
            pltpu.VMEM((obuf,), jnp.float32),
            pltpu.SemaphoreType.DMA,
            pltpu.SemaphoreType.DMA,
        ],
        compiler_params=pltpu.CompilerParams(
            use_tc_tiling_on_sc=False, needs_layout_passes=False
        ),
    )
    def emb(x_hbm, w_hbm, out_hbm, tab_v, xr0, xr1, oa, ob, soa, sob):
        wid = lax.axis_index("s") * _NC + lax.axis_index("c")
        ib0w = wid * ib_per_w
        pltpu.sync_copy(w_hbm, tab_v)
        iota = lax.iota(jnp.int32, _L)
        iota_s = iota * S
        xrs, os_, sos = [xr0, xr1], [oa, ob], [soa, sob]

        def compute(b, jb):
            def k_body(k, carry):
                jr = k // _TJ
                kk = k % _TJ
                idxv = plsc.load_gather(
                    xrs[b],
                    [
                        iota_s
                        + jnp.broadcast_to(
                            jb * _TJ + (kk * _L * S + jr), (_L,)
                        )
                    ],
                )
                a = idxv * D
                vals = [plsc.load_gather(tab_v, [a + d]) for d in range(D)]
                for d in range(D):
                    os_[b][pl.ds(d * blk + k * _L, _L)] = vals[d]
                return carry

            lax.fori_loop(0, blk // _L, k_body, 0)

        def fire_out(b, jb, ib):
            u = jb * nib + ib
            for d in range(D):
                pltpu.async_copy(
                    os_[b].at[pl.ds(d * blk, blk)],
                    out_hbm.at[pl.ds(d * (nu * blk) + u * blk, blk)],
                    sos[b],
                )

        def drain_out(b):
            pltpu.make_async_copy(
                out_hbm.at[pl.ds(0, obuf)], os_[b], sos[b]
            ).wait()

        for half in range(ib_per_w // 2):
            ib_a = ib0w + 2 * half
            for b in range(2):
                pltpu.sync_copy(
                    x_hbm.at[pl.ds((ib_a + b) * _TI * S, _TI * S)], xrs[b]
                )

            def jb_body(jb, carry):
                for b in range(2):
                    @pl.when(jnp.logical_or(jb >= 1, half >= 1))
                    def _():
                        drain_out(b)

                    compute(b, jb)
                    fire_out(b, jb, ib_a + b)
                return carry

            lax.fori_loop(0, njb, jb_body, 0)

        for b in range(2):
            drain_out(b)

    out = emb(x.reshape(N), wflat)
    return (
        out.reshape(D, njb, nib, _TJ, _TI)
        .transpose(2, 4, 1, 3, 0)
        .reshape(B, S, D)
    )
